# baseline (speedup 1.0000x reference)
import functools

import jax
import jax.numpy as jnp
from jax import lax
from jax.experimental import pallas as pl
from jax.experimental.pallas import tpu as pltpu
from jax.experimental.pallas import tpu_sc as plsc

N = 10000
NP = 10240
E = 160000
EP = 163840
NC, NS = 2, 16
NW = NC * NS
NB = EP // (NW * 128)
EB = 32
ROWS_PER_TILE = NP // NS

_f32 = jnp.float32


def _sc_mesh():
  return plsc.VectorSubcoreMesh(
      core_axis_name="c", subcore_axis_name="s", num_cores=NC,
      num_subcores=NS)


def _zero_vmem_2d(ref, nrows, ncols):
  z16 = jnp.zeros((16,), _f32)

  def body(r, carry):
    for k in range(ncols // 16):
      ref[r, pl.ds(k * 16, 16)] = z16
    return carry

  lax.fori_loop(0, nrows, body, 0)


def _fill_ones_2d(ref, nrows, ncols):
  o16 = jnp.ones((16,), _f32)

  def body(r, carry):
    for k in range(ncols // 16):
      ref[r, pl.ds(k * 16, 16)] = o16
    return carry

  lax.fori_loop(0, nrows, body, 0)


def _deg_kernel(dstm):
  @functools.partial(
      pl.kernel,
      out_type=jax.ShapeDtypeStruct((NC, NP, 128), _f32),
      mesh=_sc_mesh(),
      scratch_types=[
          pltpu.VMEM((NB, 128), jnp.int32),
          pltpu.VMEM((128, 128), _f32),
          pltpu.VMEM((128, 128), _f32),
          pltpu.VMEM_SHARED((NP, 128), _f32),
          [pltpu.SemaphoreType.DMA] * 4,
      ],
  )
  def deg_k(dstm_hbm, out_hbm, dst_v, ones_v, zb_v, acc_sp, sem):
    cid = lax.axis_index("c")
    sid = lax.axis_index("s")
    wid = cid * NS + sid
    pltpu.sync_copy(dstm_hbm.at[pl.ds(wid * NB, NB)], dst_v)
    _fill_ones_2d(ones_v, 128, 128)
    _zero_vmem_2d(zb_v, 128, 128)
    for k in range(ROWS_PER_TILE // 128):
      pltpu.sync_copy(zb_v, acc_sp.at[pl.ds(sid * ROWS_PER_TILE + k * 128, 128)])
    plsc.subcore_barrier()

    def body(g, carry):
      for j in range(4):
        pltpu.async_copy(ones_v, acc_sp.at[dst_v.at[g * 4 + j]], sem[j],
                         add=True)
      for j in range(4):
        pltpu.make_async_copy(ones_v, acc_sp.at[dst_v.at[0]], sem[j]).wait()
      return carry

    lax.fori_loop(0, NB // 4, body, 0)
    plsc.subcore_barrier()
    pltpu.sync_copy(
        acc_sp.at[pl.ds(sid * ROWS_PER_TILE, ROWS_PER_TILE)],
        out_hbm.at[cid, pl.ds(sid * ROWS_PER_TILE, ROWS_PER_TILE)])

  return deg_k(dstm)


def _scatter_kernel(zp, srcm, dstm, n_chunks, core_split):
  C = n_chunks
  NBUF = 8
  if core_split:
    assert C % 2 == 0
    nbt = (EP // NS) // EB
    out_sh = (C, NP, 128)
  else:
    nbt = (EP // NW) // EB
    out_sh = (NC, C, NP, 128)
  ngrp = nbt // NBUF
  assert ngrp % 2 == 0
  my_chunks = C // 2 if core_split else C

  @functools.partial(
      pl.kernel,
      out_type=jax.ShapeDtypeStruct(out_sh, _f32),
      mesh=_sc_mesh(),
      scratch_types=[
          [pltpu.VMEM((NBUF, EB), jnp.int32)] * 2,
          [pltpu.VMEM((NBUF, EB), jnp.int32)] * 2,
          [pltpu.VMEM((EB, 128), _f32)] * NBUF,
          pltpu.VMEM_SHARED((NP, 128), _f32),
          [pltpu.SemaphoreType.DMA] * 2,
          [pltpu.SemaphoreType.DMA] * NBUF,
          [pltpu.SemaphoreType.DMA] * NBUF,
      ],
  )
  def scat_k(zp_hbm, srcm_hbm, dstm_hbm, out_hbm, sidx, didx, rows_v,
             acc_sp, sem_i, sem_g, sem_s):
    cid = lax.axis_index("c")
    sid = lax.axis_index("s")
    if core_split:
      base = sid * nbt
    else:
      base = (cid * NS + sid) * nbt

    def idx_load(grp_mod, p):
      pltpu.async_copy(
          srcm_hbm.at[pl.ds(base + grp_mod * NBUF, NBUF)], sidx[p], sem_i[p])
      pltpu.async_copy(
          dstm_hbm.at[pl.ds(base + grp_mod * NBUF, NBUF)], didx[p], sem_i[p])

    def idx_wait(p):
      pltpu.make_async_copy(
          srcm_hbm.at[pl.ds(base, NBUF)], sidx[p], sem_i[p]).wait()
      pltpu.make_async_copy(
          dstm_hbm.at[pl.ds(base, NBUF)], didx[p], sem_i[p]).wait()

    for c in range(my_chunks):
      if core_split:
        k = 2 * c + cid
        zp_k = zp_hbm.at[k]
      else:
        k = c
        zp_k = zp_hbm.at[k]
      _zero_vmem_2d(rows_v[0], EB, 128)
      for r in range(ROWS_PER_TILE // EB):
        pltpu.sync_copy(
            rows_v[0], acc_sp.at[pl.ds(sid * ROWS_PER_TILE + r * EB, EB)])
      plsc.subcore_barrier()

      def wait_g(j):
        pltpu.make_async_copy(
            zp_k.at[sidx[0].at[j]], rows_v[j], sem_g[j]).wait()

      def start_s(j, p):
        pltpu.async_copy(rows_v[j], acc_sp.at[didx[p].at[j]], sem_s[j],
                         add=True)

      def wait_s(j):
        pltpu.make_async_copy(
            rows_v[j], acc_sp.at[didx[0].at[j]], sem_s[j]).wait()

      def start_g(j, p):
        pltpu.async_copy(zp_k.at[sidx[p].at[j]], rows_v[j], sem_g[j])

      idx_load(0, 0)
      idx_wait(0)
      for j in range(NBUF):
        start_g(j, 0)
      idx_load(1, 1)

      def super_group(m, carry):
        for j in range(NBUF):
          wait_g(j)
          start_s(j, 0)
        idx_wait(1)
        for j in range(NBUF):
          wait_s(j)
          start_g(j, 1)
        idx_load(lax.rem(2 * m + 2, ngrp), 0)
        for j in range(NBUF):
          wait_g(j)
          start_s(j, 1)
        idx_wait(0)
        for j in range(NBUF):
          wait_s(j)
          start_g(j, 0)
        idx_load(lax.rem(2 * m + 3, ngrp), 1)
        return carry

      lax.fori_loop(0, ngrp // 2, super_group, 0)
      for j in range(NBUF):
        wait_g(j)
      idx_wait(1)
      plsc.subcore_barrier()
      if core_split:
        pltpu.sync_copy(
            acc_sp.at[pl.ds(sid * ROWS_PER_TILE, ROWS_PER_TILE)],
            out_hbm.at[k, pl.ds(sid * ROWS_PER_TILE, ROWS_PER_TILE)])
      else:
        pltpu.sync_copy(
            acc_sp.at[pl.ds(sid * ROWS_PER_TILE, ROWS_PER_TILE)],
            out_hbm.at[cid, k, pl.ds(sid * ROWS_PER_TILE, ROWS_PER_TILE)])
      if c + 1 < my_chunks:
        plsc.subcore_barrier()

  return scat_k(zp, srcm, dstm)


_BM = 512


def _dinv_kernel(degp):
  def body(deg_ref, out_ref):
    d = deg_ref[0] + deg_ref[1]
    t = 1.0 + d[:, 0:1]
    out_ref[...] = lax.rsqrt(jnp.maximum(t, 1.0))

  return pl.pallas_call(
      body,
      grid=(NP // _BM,),
      in_specs=[pl.BlockSpec((NC, _BM, 128), lambda i: (0, i, 0))],
      out_specs=pl.BlockSpec((_BM, 1), lambda i: (i, 0)),
      out_shape=jax.ShapeDtypeStruct((NP, 1), _f32),
  )(degp)


def _scale_chunk_kernel(xp, dinv):
  C = xp.shape[1] // 128

  def body(x_ref, dinv_ref, out_ref):
    out_ref[0] = x_ref[...] * dinv_ref[...]

  return pl.pallas_call(
      body,
      grid=(NP // _BM, C),
      in_specs=[
          pl.BlockSpec((_BM, 128), lambda i, c: (i, c)),
          pl.BlockSpec((_BM, 1), lambda i, c: (i, 0)),
      ],
      out_specs=pl.BlockSpec((1, _BM, 128), lambda i, c: (c, i, 0)),
      out_shape=jax.ShapeDtypeStruct((C, NP, 128), _f32),
  )(xp, dinv)


def _sspec(s, C):
  if s.ndim == 4:
    return pl.BlockSpec((NC, C, _BM, 128), lambda i: (0, 0, i, 0))
  return pl.BlockSpec((C, _BM, 128), lambda i: (0, i, 0))


def _mm_fused(s, zp, dinv, W, pre_relu, post_dinv, post_relu=False):
  C_in = zp.shape[0]
  partials = s.ndim == 4
  Fo = W.shape[1]
  C_out = Fo // 128

  def body(s_ref, zp_ref, dinv_ref, w_ref, out_ref):
    acc = jnp.zeros((_BM, Fo), _f32)
    for c in range(C_in):
      t = (s_ref[0, c] + s_ref[1, c]) if partials else s_ref[c]
      hc = (t + zp_ref[c]) * dinv_ref[...]
      if pre_relu:
        hc = jnp.maximum(hc, 0.0)
      acc = acc + jnp.dot(hc, w_ref[c * 128:(c + 1) * 128, :],
                          preferred_element_type=_f32)
    if post_relu:
      acc = jnp.maximum(acc, 0.0)
    if post_dinv:
      acc = acc * dinv_ref[...]
    for co in range(C_out):
      out_ref[co] = acc[:, co * 128:(co + 1) * 128]

  return pl.pallas_call(
      body,
      grid=(NP // _BM,),
      in_specs=[
          _sspec(s, C_in),
          pl.BlockSpec((C_in, _BM, 128), lambda i: (0, i, 0)),
          pl.BlockSpec((_BM, 1), lambda i: (i, 0)),
          pl.BlockSpec((C_in * 128, Fo), lambda i: (0, 0)),
      ],
      out_specs=pl.BlockSpec((C_out, _BM, 128), lambda i: (0, i, 0)),
      out_shape=jax.ShapeDtypeStruct((C_out, NP, 128), _f32),
  )(s, zp, dinv, W)


def _mm2_fused(s, zp, dinv, W1, W2):
  C_in = zp.shape[0]
  K1 = C_in * 128
  H = W1.shape[1]
  Fo = W2.shape[1]
  C_out = Fo // 128

  def body(s_ref, zp_ref, dinv_ref, w1_ref, w2_ref, out_ref):
    acc1 = jnp.zeros((_BM, H), _f32)
    for c in range(C_in):
      uc = (s_ref[c] + zp_ref[c]) * dinv_ref[...]
      acc1 = acc1 + jnp.dot(uc, w1_ref[c * 128:(c + 1) * 128, :],
                            preferred_element_type=_f32)
    h = jnp.maximum(acc1, 0.0)
    acc2 = jnp.dot(h, w2_ref[...], preferred_element_type=_f32)
    acc2 = acc2 * dinv_ref[...]
    for co in range(C_out):
      out_ref[co] = acc2[:, co * 128:(co + 1) * 128]

  return pl.pallas_call(
      body,
      grid=(NP // _BM,),
      in_specs=[
          _sspec(s, C_in),
          pl.BlockSpec((C_in, _BM, 128), lambda i: (0, i, 0)),
          pl.BlockSpec((_BM, 1), lambda i: (i, 0)),
          pl.BlockSpec((K1, H), lambda i: (0, 0)),
          pl.BlockSpec((H, Fo), lambda i: (0, 0)),
      ],
      out_specs=pl.BlockSpec((C_out, _BM, 128), lambda i: (0, i, 0)),
      out_shape=jax.ShapeDtypeStruct((C_out, NP, 128), _f32),
  )(s, zp, dinv, W1, W2)


def _combine_kernel(s, zp, dinv, relu):
  C = zp.shape[0]
  partials = s.ndim == 4

  def body(s_ref, zp_ref, dinv_ref, out_ref):
    if partials:
      t = s_ref[0, 0] + s_ref[1, 0]
    else:
      t = s_ref[0]
    v = (t + zp_ref[0]) * dinv_ref[...]
    if relu:
      v = jnp.maximum(v, 0.0)
    out_ref[0] = v

  s_spec = (
      pl.BlockSpec((NC, 1, _BM, 128), lambda i, c: (0, c, i, 0)) if partials
      else pl.BlockSpec((1, _BM, 128), lambda i, c: (c, i, 0)))

  return pl.pallas_call(
      body,
      grid=(NP // _BM, C),
      in_specs=[
          s_spec,
          pl.BlockSpec((1, _BM, 128), lambda i, c: (c, i, 0)),
          pl.BlockSpec((_BM, 1), lambda i, c: (i, 0)),
      ],
      out_specs=pl.BlockSpec((1, _BM, 128), lambda i, c: (c, i, 0)),
      out_shape=jax.ShapeDtypeStruct((C, NP, 128), _f32),
  )(s, zp, dinv)


def _pad2(a, rows, cols):
  return jnp.pad(a, ((0, rows - a.shape[0]), (0, cols - a.shape[1])))


def kernel(x, edge_index, W1, W2, W3, W4):
  xp = _pad2(x, NP, 256)
  W1p = _pad2(W1, 256, 1024)
  W2p = _pad2(W2, 1024, 512)
  W3p = _pad2(W3, 512, 128)
  W4p = _pad2(W4, 128, 128)

  pad = jnp.full((EP - E,), NP - 1, jnp.int32)
  sr = jnp.concatenate([edge_index[0], pad])
  ds = jnp.concatenate([edge_index[1], pad])
  srcm = sr.reshape(EP // EB, EB)
  dstm = ds.reshape(EP // EB, EB)
  dstm128 = ds.reshape(EP // 128, 128)

  degp = _deg_kernel(dstm128)
  dinv = _dinv_kernel(degp)

  z1 = _scale_chunk_kernel(xp, dinv)
  s1 = _scatter_kernel(z1, srcm, dstm, 2, core_split=True)

  z2 = _mm2_fused(s1, z1, dinv, W1p, W2p)
  s2 = _scatter_kernel(z2, srcm, dstm, 4, core_split=True)

  z3 = _mm_fused(s2, z2, dinv, W3p, pre_relu=True, post_dinv=True)
  s3 = _scatter_kernel(z3, srcm, dstm, 1, core_split=False)

  z4 = _mm_fused(s3, z3, dinv, W4p, pre_relu=True, post_dinv=True)
  s4 = _scatter_kernel(z4, srcm, dstm, 1, core_split=False)
  oc = _combine_kernel(s4, z4, dinv, relu=False)

  return oc[0, :N, :40]


# speedup vs baseline: 1.0238x; 1.0238x over previous
import functools

import jax
import jax.numpy as jnp
from jax import lax
from jax.experimental import pallas as pl
from jax.experimental.pallas import tpu as pltpu
from jax.experimental.pallas import tpu_sc as plsc

N = 10000
NP = 10240
E = 160000
EP = 163840
NC, NS = 2, 16
NW = NC * NS
NB = EP // (NW * 128)
EB = 32
ROWS_PER_TILE = NP // NS

_f32 = jnp.float32


def _sc_mesh():
  return plsc.VectorSubcoreMesh(
      core_axis_name="c", subcore_axis_name="s", num_cores=NC,
      num_subcores=NS)


def _zero_vmem_2d(ref, nrows, ncols):
  z16 = jnp.zeros((16,), _f32)

  def body(r, carry):
    for k in range(ncols // 16):
      ref[r, pl.ds(k * 16, 16)] = z16
    return carry

  lax.fori_loop(0, nrows, body, 0)


def _fill_ones_2d(ref, nrows, ncols):
  o16 = jnp.ones((16,), _f32)

  def body(r, carry):
    for k in range(ncols // 16):
      ref[r, pl.ds(k * 16, 16)] = o16
    return carry

  lax.fori_loop(0, nrows, body, 0)


def _deg_kernel(dstm):
  @functools.partial(
      pl.kernel,
      out_type=jax.ShapeDtypeStruct((NC, NP, 128), _f32),
      mesh=_sc_mesh(),
      scratch_types=[
          pltpu.VMEM((NB, 128), jnp.int32),
          pltpu.VMEM((128, 128), _f32),
          pltpu.VMEM((128, 128), _f32),
          pltpu.VMEM_SHARED((NP, 128), _f32),
          [pltpu.SemaphoreType.DMA] * 4,
      ],
  )
  def deg_k(dstm_hbm, out_hbm, dst_v, ones_v, zb_v, acc_sp, sem):
    cid = lax.axis_index("c")
    sid = lax.axis_index("s")
    wid = cid * NS + sid
    pltpu.sync_copy(dstm_hbm.at[pl.ds(wid * NB, NB)], dst_v)
    _fill_ones_2d(ones_v, 128, 128)
    _zero_vmem_2d(zb_v, 128, 128)
    for k in range(ROWS_PER_TILE // 128):
      pltpu.sync_copy(zb_v, acc_sp.at[pl.ds(sid * ROWS_PER_TILE + k * 128, 128)])
    plsc.subcore_barrier()

    def body(g, carry):
      for j in range(4):
        pltpu.async_copy(ones_v, acc_sp.at[dst_v.at[g * 4 + j]], sem[j],
                         add=True)
      for j in range(4):
        pltpu.make_async_copy(ones_v, acc_sp.at[dst_v.at[0]], sem[j]).wait()
      return carry

    lax.fori_loop(0, NB // 4, body, 0)
    plsc.subcore_barrier()
    pltpu.sync_copy(
        acc_sp.at[pl.ds(sid * ROWS_PER_TILE, ROWS_PER_TILE)],
        out_hbm.at[cid, pl.ds(sid * ROWS_PER_TILE, ROWS_PER_TILE)])

  return deg_k(dstm)


def _scatter_kernel(zp, srcm, dstm, n_chunks, core_split):
  C = n_chunks
  NBUF = 8
  if core_split:
    assert C % 2 == 0
    nbt = (EP // NS) // EB
    out_sh = (C, NP, 128)
  else:
    nbt = (EP // NW) // EB
    out_sh = (NC, C, NP, 128)
  ngrp = nbt // NBUF
  assert ngrp % 2 == 0
  my_chunks = C // 2 if core_split else C

  @functools.partial(
      pl.kernel,
      out_type=jax.ShapeDtypeStruct(out_sh, _f32),
      mesh=_sc_mesh(),
      scratch_types=[
          [pltpu.VMEM((NBUF, EB), jnp.int32)] * 2,
          [pltpu.VMEM((NBUF, EB), jnp.int32)] * 2,
          [pltpu.VMEM((EB, 128), _f32)] * NBUF,
          pltpu.VMEM_SHARED((NP, 128), _f32),
          [pltpu.SemaphoreType.DMA] * 2,
          [pltpu.SemaphoreType.DMA] * NBUF,
          [pltpu.SemaphoreType.DMA] * NBUF,
      ],
  )
  def scat_k(zp_hbm, srcm_hbm, dstm_hbm, out_hbm, sidx, didx, rows_v,
             acc_sp, sem_i, sem_g, sem_s):
    cid = lax.axis_index("c")
    sid = lax.axis_index("s")
    if core_split:
      base = sid * nbt
    else:
      base = (cid * NS + sid) * nbt

    def idx_load(grp_mod, p):
      pltpu.async_copy(
          srcm_hbm.at[pl.ds(base + grp_mod * NBUF, NBUF)], sidx[p], sem_i[p])
      pltpu.async_copy(
          dstm_hbm.at[pl.ds(base + grp_mod * NBUF, NBUF)], didx[p], sem_i[p])

    def idx_wait(p):
      pltpu.make_async_copy(
          srcm_hbm.at[pl.ds(base, NBUF)], sidx[p], sem_i[p]).wait()
      pltpu.make_async_copy(
          dstm_hbm.at[pl.ds(base, NBUF)], didx[p], sem_i[p]).wait()

    for c in range(my_chunks):
      if core_split:
        k = 2 * c + cid
        zp_k = zp_hbm.at[k]
      else:
        k = c
        zp_k = zp_hbm.at[k]
      _zero_vmem_2d(rows_v[0], EB, 128)
      for r in range(ROWS_PER_TILE // EB):
        pltpu.sync_copy(
            rows_v[0], acc_sp.at[pl.ds(sid * ROWS_PER_TILE + r * EB, EB)])
      plsc.subcore_barrier()

      def wait_g(j):
        pltpu.make_async_copy(
            zp_k.at[sidx[0].at[j]], rows_v[j], sem_g[j]).wait()

      def start_s(j, p):
        pltpu.async_copy(rows_v[j], acc_sp.at[didx[p].at[j]], sem_s[j],
                         add=True)

      def wait_s(j):
        pltpu.make_async_copy(
            rows_v[j], acc_sp.at[didx[0].at[j]], sem_s[j]).wait()

      def start_g(j, p):
        pltpu.async_copy(zp_k.at[sidx[p].at[j]], rows_v[j], sem_g[j])

      idx_load(0, 0)
      idx_wait(0)
      for j in range(NBUF):
        start_g(j, 0)
      idx_load(1, 1)

      def super_group(m, carry):
        for j in range(NBUF):
          wait_g(j)
          start_s(j, 0)
        idx_wait(1)
        for j in range(NBUF):
          wait_s(j)
          start_g(j, 1)
        idx_load(lax.rem(2 * m + 2, ngrp), 0)
        for j in range(NBUF):
          wait_g(j)
          start_s(j, 1)
        idx_wait(0)
        for j in range(NBUF):
          wait_s(j)
          start_g(j, 0)
        idx_load(lax.rem(2 * m + 3, ngrp), 1)
        return carry

      lax.fori_loop(0, ngrp // 2, super_group, 0)
      for j in range(NBUF):
        wait_g(j)
      idx_wait(1)
      plsc.subcore_barrier()
      if core_split:
        pltpu.sync_copy(
            acc_sp.at[pl.ds(sid * ROWS_PER_TILE, ROWS_PER_TILE)],
            out_hbm.at[k, pl.ds(sid * ROWS_PER_TILE, ROWS_PER_TILE)])
      else:
        pltpu.sync_copy(
            acc_sp.at[pl.ds(sid * ROWS_PER_TILE, ROWS_PER_TILE)],
            out_hbm.at[cid, k, pl.ds(sid * ROWS_PER_TILE, ROWS_PER_TILE)])
      if c + 1 < my_chunks:
        plsc.subcore_barrier()

  return scat_k(zp, srcm, dstm)


_BM = 1024


def _dinv_kernel(degp):
  def body(deg_ref, out_ref):
    d = deg_ref[0] + deg_ref[1]
    t = 1.0 + d[:, 0:1]
    out_ref[...] = lax.rsqrt(jnp.maximum(t, 1.0))

  return pl.pallas_call(
      body,
      grid=(NP // _BM,),
      in_specs=[pl.BlockSpec((NC, _BM, 128), lambda i: (0, i, 0))],
      out_specs=pl.BlockSpec((_BM, 1), lambda i: (i, 0)),
      out_shape=jax.ShapeDtypeStruct((NP, 1), _f32),
  )(degp)


def _scale_chunk_kernel(xp, dinv):
  C = xp.shape[1] // 128

  def body(x_ref, dinv_ref, out_ref):
    out_ref[0] = x_ref[...] * dinv_ref[...]

  return pl.pallas_call(
      body,
      grid=(NP // _BM, C),
      in_specs=[
          pl.BlockSpec((_BM, 128), lambda i, c: (i, c)),
          pl.BlockSpec((_BM, 1), lambda i, c: (i, 0)),
      ],
      out_specs=pl.BlockSpec((1, _BM, 128), lambda i, c: (c, i, 0)),
      out_shape=jax.ShapeDtypeStruct((C, NP, 128), _f32),
  )(xp, dinv)


def _sspec(s, C):
  if s.ndim == 4:
    return pl.BlockSpec((NC, C, _BM, 128), lambda i: (0, 0, i, 0))
  return pl.BlockSpec((C, _BM, 128), lambda i: (0, i, 0))


def _mm_fused(s, zp, dinv, W, pre_relu, post_dinv, post_relu=False):
  C_in = zp.shape[0]
  partials = s.ndim == 4
  Fo = W.shape[1]
  C_out = Fo // 128

  def body(s_ref, zp_ref, dinv_ref, w_ref, out_ref):
    acc = jnp.zeros((_BM, Fo), _f32)
    for c in range(C_in):
      t = (s_ref[0, c] + s_ref[1, c]) if partials else s_ref[c]
      hc = (t + zp_ref[c]) * dinv_ref[...]
      if pre_relu:
        hc = jnp.maximum(hc, 0.0)
      acc = acc + jnp.dot(hc, w_ref[c * 128:(c + 1) * 128, :],
                          preferred_element_type=_f32)
    if post_relu:
      acc = jnp.maximum(acc, 0.0)
    if post_dinv:
      acc = acc * dinv_ref[...]
    for co in range(C_out):
      out_ref[co] = acc[:, co * 128:(co + 1) * 128]

  return pl.pallas_call(
      body,
      grid=(NP // _BM,),
      in_specs=[
          _sspec(s, C_in),
          pl.BlockSpec((C_in, _BM, 128), lambda i: (0, i, 0)),
          pl.BlockSpec((_BM, 1), lambda i: (i, 0)),
          pl.BlockSpec((C_in * 128, Fo), lambda i: (0, 0)),
      ],
      out_specs=pl.BlockSpec((C_out, _BM, 128), lambda i: (0, i, 0)),
      out_shape=jax.ShapeDtypeStruct((C_out, NP, 128), _f32),
  )(s, zp, dinv, W)


def _mm2_fused(s, zp, dinv, W1, W2):
  C_in = zp.shape[0]
  K1 = C_in * 128
  H = W1.shape[1]
  Fo = W2.shape[1]
  C_out = Fo // 128

  def body(s_ref, zp_ref, dinv_ref, w1_ref, w2_ref, out_ref):
    acc1 = jnp.zeros((_BM, H), _f32)
    for c in range(C_in):
      uc = (s_ref[c] + zp_ref[c]) * dinv_ref[...]
      acc1 = acc1 + jnp.dot(uc, w1_ref[c * 128:(c + 1) * 128, :],
                            preferred_element_type=_f32)
    h = jnp.maximum(acc1, 0.0)
    acc2 = jnp.dot(h, w2_ref[...], preferred_element_type=_f32)
    acc2 = acc2 * dinv_ref[...]
    for co in range(C_out):
      out_ref[co] = acc2[:, co * 128:(co + 1) * 128]

  return pl.pallas_call(
      body,
      grid=(NP // _BM,),
      in_specs=[
          _sspec(s, C_in),
          pl.BlockSpec((C_in, _BM, 128), lambda i: (0, i, 0)),
          pl.BlockSpec((_BM, 1), lambda i: (i, 0)),
          pl.BlockSpec((K1, H), lambda i: (0, 0)),
          pl.BlockSpec((H, Fo), lambda i: (0, 0)),
      ],
      out_specs=pl.BlockSpec((C_out, _BM, 128), lambda i: (0, i, 0)),
      out_shape=jax.ShapeDtypeStruct((C_out, NP, 128), _f32),
  )(s, zp, dinv, W1, W2)


def _combine_kernel(s, zp, dinv, relu):
  C = zp.shape[0]
  partials = s.ndim == 4

  def body(s_ref, zp_ref, dinv_ref, out_ref):
    if partials:
      t = s_ref[0, 0] + s_ref[1, 0]
    else:
      t = s_ref[0]
    v = (t + zp_ref[0]) * dinv_ref[...]
    if relu:
      v = jnp.maximum(v, 0.0)
    out_ref[0] = v

  s_spec = (
      pl.BlockSpec((NC, 1, _BM, 128), lambda i, c: (0, c, i, 0)) if partials
      else pl.BlockSpec((1, _BM, 128), lambda i, c: (c, i, 0)))

  return pl.pallas_call(
      body,
      grid=(NP // _BM, C),
      in_specs=[
          s_spec,
          pl.BlockSpec((1, _BM, 128), lambda i, c: (c, i, 0)),
          pl.BlockSpec((_BM, 1), lambda i, c: (i, 0)),
      ],
      out_specs=pl.BlockSpec((1, _BM, 128), lambda i, c: (c, i, 0)),
      out_shape=jax.ShapeDtypeStruct((C, NP, 128), _f32),
  )(s, zp, dinv)


def _pad2(a, rows, cols):
  return jnp.pad(a, ((0, rows - a.shape[0]), (0, cols - a.shape[1])))


def kernel(x, edge_index, W1, W2, W3, W4):
  xp = _pad2(x, NP, 256)
  W1p = _pad2(W1, 256, 1024)
  W2p = _pad2(W2, 1024, 512)
  W3p = _pad2(W3, 512, 128)
  W4p = _pad2(W4, 128, 128)

  pad = jnp.full((EP - E,), NP - 1, jnp.int32)
  sr = jnp.concatenate([edge_index[0], pad])
  ds = jnp.concatenate([edge_index[1], pad])
  srcm = sr.reshape(EP // EB, EB)
  dstm = ds.reshape(EP // EB, EB)
  dstm128 = ds.reshape(EP // 128, 128)

  degp = _deg_kernel(dstm128)
  dinv = _dinv_kernel(degp)

  z1 = _scale_chunk_kernel(xp, dinv)
  s1 = _scatter_kernel(z1, srcm, dstm, 2, core_split=True)

  z2 = _mm2_fused(s1, z1, dinv, W1p, W2p)
  s2 = _scatter_kernel(z2, srcm, dstm, 4, core_split=True)

  z3 = _mm_fused(s2, z2, dinv, W3p, pre_relu=True, post_dinv=True)
  s3 = _scatter_kernel(z3, srcm, dstm, 1, core_split=False)

  z4 = _mm_fused(s3, z3, dinv, W4p, pre_relu=True, post_dinv=True)
  s4 = _scatter_kernel(z4, srcm, dstm, 1, core_split=False)
  oc = _combine_kernel(s4, z4, dinv, relu=False)

  return oc[0, :N, :40]


# speedup vs baseline: 1.0329x; 1.0089x over previous
import functools

import jax
import jax.numpy as jnp
from jax import lax
from jax.experimental import pallas as pl
from jax.experimental.pallas import tpu as pltpu
from jax.experimental.pallas import tpu_sc as plsc

N = 10000
NP = 10240
E = 160000
EP = 163840
NC, NS = 2, 16
NW = NC * NS
NB = EP // (NW * 128)
EB = 32
ROWS_PER_TILE = NP // NS

_f32 = jnp.float32


def _sc_mesh():
  return plsc.VectorSubcoreMesh(
      core_axis_name="c", subcore_axis_name="s", num_cores=NC,
      num_subcores=NS)


def _zero_vmem_2d(ref, nrows, ncols):
  z16 = jnp.zeros((16,), _f32)

  def body(r, carry):
    for k in range(ncols // 16):
      ref[r, pl.ds(k * 16, 16)] = z16
    return carry

  lax.fori_loop(0, nrows, body, 0)


def _fill_ones_2d(ref, nrows, ncols):
  o16 = jnp.ones((16,), _f32)

  def body(r, carry):
    for k in range(ncols // 16):
      ref[r, pl.ds(k * 16, 16)] = o16
    return carry

  lax.fori_loop(0, nrows, body, 0)


def _deg_kernel(dstm):
  @functools.partial(
      pl.kernel,
      out_type=jax.ShapeDtypeStruct((NC, NP, 128), _f32),
      mesh=_sc_mesh(),
      scratch_types=[
          pltpu.VMEM((NB, 128), jnp.int32),
          pltpu.VMEM((128, 128), _f32),
          pltpu.VMEM((128, 128), _f32),
          pltpu.VMEM_SHARED((NP, 128), _f32),
          [pltpu.SemaphoreType.DMA] * 4,
      ],
  )
  def deg_k(dstm_hbm, out_hbm, dst_v, ones_v, zb_v, acc_sp, sem):
    cid = lax.axis_index("c")
    sid = lax.axis_index("s")
    wid = cid * NS + sid
    pltpu.sync_copy(dstm_hbm.at[pl.ds(wid * NB, NB)], dst_v)
    _fill_ones_2d(ones_v, 128, 128)
    _zero_vmem_2d(zb_v, 128, 128)
    for k in range(ROWS_PER_TILE // 128):
      pltpu.sync_copy(zb_v, acc_sp.at[pl.ds(sid * ROWS_PER_TILE + k * 128, 128)])
    plsc.subcore_barrier()

    def body(g, carry):
      for j in range(4):
        pltpu.async_copy(ones_v, acc_sp.at[dst_v.at[g * 4 + j]], sem[j],
                         add=True)
      for j in range(4):
        pltpu.make_async_copy(ones_v, acc_sp.at[dst_v.at[0]], sem[j]).wait()
      return carry

    lax.fori_loop(0, NB // 4, body, 0)
    plsc.subcore_barrier()
    pltpu.sync_copy(
        acc_sp.at[pl.ds(sid * ROWS_PER_TILE, ROWS_PER_TILE)],
        out_hbm.at[cid, pl.ds(sid * ROWS_PER_TILE, ROWS_PER_TILE)])

  return deg_k(dstm)


def _scatter_kernel(zp, srcm, dstm, n_chunks, core_split):
  C = n_chunks
  NBUF = 8
  if core_split:
    assert C % 2 == 0
    nbt = (EP // NS) // EB
    out_sh = (C, NP, 128)
  else:
    nbt = (EP // NW) // EB
    out_sh = (NC, C, NP, 128)
  ngrp = nbt // NBUF
  assert ngrp % 2 == 0
  my_chunks = C // 2 if core_split else C

  @functools.partial(
      pl.kernel,
      out_type=jax.ShapeDtypeStruct(out_sh, _f32),
      mesh=_sc_mesh(),
      scratch_types=[
          [pltpu.VMEM((NBUF, EB), jnp.int32)] * 2,
          [pltpu.VMEM((NBUF, EB), jnp.int32)] * 2,
          [pltpu.VMEM((EB, 128), _f32)] * NBUF,
          pltpu.VMEM_SHARED((NP, 128), _f32),
          [pltpu.SemaphoreType.DMA] * 2,
          [pltpu.SemaphoreType.DMA] * NBUF,
          [pltpu.SemaphoreType.DMA] * NBUF,
      ],
  )
  def scat_k(zp_hbm, srcm_hbm, dstm_hbm, out_hbm, sidx, didx, rows_v,
             acc_sp, sem_i, sem_g, sem_s):
    cid = lax.axis_index("c")
    sid = lax.axis_index("s")
    if core_split:
      base = sid * nbt
    else:
      base = (cid * NS + sid) * nbt

    def idx_load(grp_mod, p):
      pltpu.async_copy(
          srcm_hbm.at[pl.ds(base + grp_mod * NBUF, NBUF)], sidx[p], sem_i[p])
      pltpu.async_copy(
          dstm_hbm.at[pl.ds(base + grp_mod * NBUF, NBUF)], didx[p], sem_i[p])

    def idx_wait(p):
      pltpu.make_async_copy(
          srcm_hbm.at[pl.ds(base, NBUF)], sidx[p], sem_i[p]).wait()
      pltpu.make_async_copy(
          dstm_hbm.at[pl.ds(base, NBUF)], didx[p], sem_i[p]).wait()

    for c in range(my_chunks):
      if core_split:
        k = 2 * c + cid
        zp_k = zp_hbm.at[k]
      else:
        k = c
        zp_k = zp_hbm.at[k]
      _zero_vmem_2d(rows_v[0], EB, 128)
      for r in range(ROWS_PER_TILE // EB):
        pltpu.sync_copy(
            rows_v[0], acc_sp.at[pl.ds(sid * ROWS_PER_TILE + r * EB, EB)])
      plsc.subcore_barrier()

      def wait_g(j):
        pltpu.make_async_copy(
            zp_k.at[sidx[0].at[j]], rows_v[j], sem_g[j]).wait()

      def start_s(j, p):
        pltpu.async_copy(rows_v[j], acc_sp.at[didx[p].at[j]], sem_s[j],
                         add=True)

      def wait_s(j):
        pltpu.make_async_copy(
            rows_v[j], acc_sp.at[didx[0].at[j]], sem_s[j]).wait()

      def start_g(j, p):
        pltpu.async_copy(zp_k.at[sidx[p].at[j]], rows_v[j], sem_g[j])

      idx_load(0, 0)
      idx_wait(0)
      for j in range(NBUF):
        start_g(j, 0)
      idx_load(1, 1)

      def super_group(m, carry):
        for j in range(NBUF):
          wait_g(j)
          start_s(j, 0)
        idx_wait(1)
        for j in range(NBUF):
          wait_s(j)
          start_g(j, 1)
        idx_load(lax.rem(2 * m + 2, ngrp), 0)
        for j in range(NBUF):
          wait_g(j)
          start_s(j, 1)
        idx_wait(0)
        for j in range(NBUF):
          wait_s(j)
          start_g(j, 0)
        idx_load(lax.rem(2 * m + 3, ngrp), 1)
        return carry

      lax.fori_loop(0, ngrp // 2, super_group, 0)
      for j in range(NBUF):
        wait_g(j)
      idx_wait(1)
      plsc.subcore_barrier()
      if core_split:
        pltpu.sync_copy(
            acc_sp.at[pl.ds(sid * ROWS_PER_TILE, ROWS_PER_TILE)],
            out_hbm.at[k, pl.ds(sid * ROWS_PER_TILE, ROWS_PER_TILE)])
      else:
        pltpu.sync_copy(
            acc_sp.at[pl.ds(sid * ROWS_PER_TILE, ROWS_PER_TILE)],
            out_hbm.at[cid, k, pl.ds(sid * ROWS_PER_TILE, ROWS_PER_TILE)])
      if c + 1 < my_chunks:
        plsc.subcore_barrier()

  return scat_k(zp, srcm, dstm)


_BM = 2048


def _dinv_kernel(degp):
  def body(deg_ref, out_ref):
    d = deg_ref[0] + deg_ref[1]
    t = 1.0 + d[:, 0:1]
    out_ref[...] = lax.rsqrt(jnp.maximum(t, 1.0))

  return pl.pallas_call(
      body,
      grid=(NP // _BM,),
      in_specs=[pl.BlockSpec((NC, _BM, 128), lambda i: (0, i, 0))],
      out_specs=pl.BlockSpec((_BM, 1), lambda i: (i, 0)),
      out_shape=jax.ShapeDtypeStruct((NP, 1), _f32),
  )(degp)


def _scale_chunk_kernel(xp, dinv):
  C = xp.shape[1] // 128

  def body(x_ref, dinv_ref, out_ref):
    out_ref[0] = x_ref[...] * dinv_ref[...]

  return pl.pallas_call(
      body,
      grid=(NP // _BM, C),
      in_specs=[
          pl.BlockSpec((_BM, 128), lambda i, c: (i, c)),
          pl.BlockSpec((_BM, 1), lambda i, c: (i, 0)),
      ],
      out_specs=pl.BlockSpec((1, _BM, 128), lambda i, c: (c, i, 0)),
      out_shape=jax.ShapeDtypeStruct((C, NP, 128), _f32),
  )(xp, dinv)


def _sspec(s, C):
  if s.ndim == 4:
    return pl.BlockSpec((NC, C, _BM, 128), lambda i: (0, 0, i, 0))
  return pl.BlockSpec((C, _BM, 128), lambda i: (0, i, 0))


def _mm_fused(s, zp, dinv, W, pre_relu, post_dinv, post_relu=False):
  C_in = zp.shape[0]
  partials = s.ndim == 4
  Fo = W.shape[1]
  C_out = Fo // 128

  def body(s_ref, zp_ref, dinv_ref, w_ref, out_ref):
    acc = jnp.zeros((_BM, Fo), _f32)
    for c in range(C_in):
      t = (s_ref[0, c] + s_ref[1, c]) if partials else s_ref[c]
      hc = (t + zp_ref[c]) * dinv_ref[...]
      if pre_relu:
        hc = jnp.maximum(hc, 0.0)
      acc = acc + jnp.dot(hc, w_ref[c * 128:(c + 1) * 128, :],
                          preferred_element_type=_f32)
    if post_relu:
      acc = jnp.maximum(acc, 0.0)
    if post_dinv:
      acc = acc * dinv_ref[...]
    for co in range(C_out):
      out_ref[co] = acc[:, co * 128:(co + 1) * 128]

  return pl.pallas_call(
      body,
      grid=(NP // _BM,),
      in_specs=[
          _sspec(s, C_in),
          pl.BlockSpec((C_in, _BM, 128), lambda i: (0, i, 0)),
          pl.BlockSpec((_BM, 1), lambda i: (i, 0)),
          pl.BlockSpec((C_in * 128, Fo), lambda i: (0, 0)),
      ],
      out_specs=pl.BlockSpec((C_out, _BM, 128), lambda i: (0, i, 0)),
      out_shape=jax.ShapeDtypeStruct((C_out, NP, 128), _f32),
  )(s, zp, dinv, W)


def _mm2_fused(s, zp, dinv, W1, W2):
  C_in = zp.shape[0]
  K1 = C_in * 128
  H = W1.shape[1]
  Fo = W2.shape[1]
  C_out = Fo // 128

  def body(s_ref, zp_ref, dinv_ref, w1_ref, w2_ref, out_ref):
    acc1 = jnp.zeros((_BM, H), _f32)
    for c in range(C_in):
      uc = (s_ref[c] + zp_ref[c]) * dinv_ref[...]
      acc1 = acc1 + jnp.dot(uc, w1_ref[c * 128:(c + 1) * 128, :],
                            preferred_element_type=_f32)
    h = jnp.maximum(acc1, 0.0)
    acc2 = jnp.dot(h, w2_ref[...], preferred_element_type=_f32)
    acc2 = acc2 * dinv_ref[...]
    for co in range(C_out):
      out_ref[co] = acc2[:, co * 128:(co + 1) * 128]

  return pl.pallas_call(
      body,
      grid=(NP // _BM,),
      in_specs=[
          _sspec(s, C_in),
          pl.BlockSpec((C_in, _BM, 128), lambda i: (0, i, 0)),
          pl.BlockSpec((_BM, 1), lambda i: (i, 0)),
          pl.BlockSpec((K1, H), lambda i: (0, 0)),
          pl.BlockSpec((H, Fo), lambda i: (0, 0)),
      ],
      out_specs=pl.BlockSpec((C_out, _BM, 128), lambda i: (0, i, 0)),
      out_shape=jax.ShapeDtypeStruct((C_out, NP, 128), _f32),
  )(s, zp, dinv, W1, W2)


def _combine_kernel(s, zp, dinv, relu):
  C = zp.shape[0]
  partials = s.ndim == 4

  def body(s_ref, zp_ref, dinv_ref, out_ref):
    if partials:
      t = s_ref[0, 0] + s_ref[1, 0]
    else:
      t = s_ref[0]
    v = (t + zp_ref[0]) * dinv_ref[...]
    if relu:
      v = jnp.maximum(v, 0.0)
    out_ref[0] = v

  s_spec = (
      pl.BlockSpec((NC, 1, _BM, 128), lambda i, c: (0, c, i, 0)) if partials
      else pl.BlockSpec((1, _BM, 128), lambda i, c: (c, i, 0)))

  return pl.pallas_call(
      body,
      grid=(NP // _BM, C),
      in_specs=[
          s_spec,
          pl.BlockSpec((1, _BM, 128), lambda i, c: (c, i, 0)),
          pl.BlockSpec((_BM, 1), lambda i, c: (i, 0)),
      ],
      out_specs=pl.BlockSpec((1, _BM, 128), lambda i, c: (c, i, 0)),
      out_shape=jax.ShapeDtypeStruct((C, NP, 128), _f32),
  )(s, zp, dinv)


def _pad2(a, rows, cols):
  return jnp.pad(a, ((0, rows - a.shape[0]), (0, cols - a.shape[1])))


def kernel(x, edge_index, W1, W2, W3, W4):
  xp = _pad2(x, NP, 256)
  W1p = _pad2(W1, 256, 1024)
  W2p = _pad2(W2, 1024, 512)
  W3p = _pad2(W3, 512, 128)
  W4p = _pad2(W4, 128, 128)

  pad = jnp.full((EP - E,), NP - 1, jnp.int32)
  sr = jnp.concatenate([edge_index[0], pad])
  ds = jnp.concatenate([edge_index[1], pad])
  srcm = sr.reshape(EP // EB, EB)
  dstm = ds.reshape(EP // EB, EB)
  dstm128 = ds.reshape(EP // 128, 128)

  degp = _deg_kernel(dstm128)
  dinv = _dinv_kernel(degp)

  z1 = _scale_chunk_kernel(xp, dinv)
  s1 = _scatter_kernel(z1, srcm, dstm, 2, core_split=True)

  z2 = _mm2_fused(s1, z1, dinv, W1p, W2p)
  s2 = _scatter_kernel(z2, srcm, dstm, 4, core_split=True)

  z3 = _mm_fused(s2, z2, dinv, W3p, pre_relu=True, post_dinv=True)
  s3 = _scatter_kernel(z3, srcm, dstm, 1, core_split=False)

  z4 = _mm_fused(s3, z3, dinv, W4p, pre_relu=True, post_dinv=True)
  s4 = _scatter_kernel(z4, srcm, dstm, 1, core_split=False)
  oc = _combine_kernel(s4, z4, dinv, relu=False)

  return oc[0, :N, :40]
